# baseline (speedup 1.0000x reference)
import jax
import jax.numpy as jnp
from jax import lax
from jax.experimental import pallas as pl
from jax.experimental.pallas import tpu as pltpu
from jax.experimental.pallas import tpu_sc as plsc

D_MODEL = 64
SCALE = 8.0
NC, NS = 2, 16
NW = NC * NS
HB = 200 // 8
BB = 4096 // 128
CHUNK = 512


def _gather_body(src4_hbm, table_hbm, mid_hbm, idx_all, rows2, sg0, sg1, ss0, ss1):
    wid = lax.axis_index("s") * NC + lax.axis_index("c")
    sg = (sg0, sg1)
    ss = (ss0, ss1)

    pltpu.sync_copy(src4_hbm.at[pl.ds(0, HB), wid], idx_all)

    def fire_gather(k, half, b):
        for j in range(4):
            pltpu.async_copy(
                table_hbm.at[idx_all.at[k, half * 4 + j]],
                rows2.at[b, pl.ds(j * 128, 128)],
                sg[b],
            )

    def wait_gather(b):
        pltpu.make_async_copy(
            table_hbm.at[pl.ds(0, CHUNK)], rows2.at[b], sg[b]
        ).wait()

    def fire_store(c, b):
        pltpu.async_copy(
            rows2.at[b], mid_hbm.at[wid, pl.ds(c * CHUNK, CHUNK)], ss[b]
        )

    def wait_store(b):
        pltpu.make_async_copy(
            rows2.at[b], mid_hbm.at[wid, pl.ds(0, CHUNK)], ss[b]
        ).wait()

    fire_gather(0, 0, 0)
    fire_gather(0, 1, 1)

    @pl.loop(0, HB)
    def _pair(k):
        wait_gather(0)
        fire_store(2 * k, 0)

        @pl.when(k < HB - 1)
        def _():
            wait_store(0)
            fire_gather(k + 1, 0, 0)

        wait_gather(1)
        fire_store(2 * k + 1, 1)

        @pl.when(k < HB - 1)
        def _():
            wait_store(1)
            fire_gather(k + 1, 1, 1)

    wait_store(0)
    wait_store(1)


def _tc_transpose_body(x_ref, y_ref):
    x = x_ref[0, 0]
    xe = x[:, :, 0:64]
    xo = x[:, :, 64:128]
    te = jnp.transpose(xe, (0, 2, 1))
    to = jnp.transpose(xo, (0, 2, 1))
    t = jnp.stack([te, to], axis=-1)
    y = t.reshape(8, 64, 128) * SCALE
    y_ref[...] = y.reshape(8, 8, 1, 8, 128)


def kernel(src, table):
    src4 = (
        src.astype(jnp.int32)
        .swapaxes(0, 1)
        .reshape(HB, 8, BB, 128)
        .transpose(0, 2, 1, 3)
    )
    mesh = plsc.VectorSubcoreMesh(
        core_axis_name="c", subcore_axis_name="s", num_cores=NC, num_subcores=NS
    )
    mid = pl.kernel(
        _gather_body,
        out_type=jax.ShapeDtypeStruct((NW, HB * 1024, D_MODEL), jnp.float32),
        mesh=mesh,
        scratch_types=[
            pltpu.VMEM((HB, 8, 128), jnp.int32),
            pltpu.VMEM((2, CHUNK, D_MODEL), jnp.float32),
            pltpu.SemaphoreType.DMA,
            pltpu.SemaphoreType.DMA,
            pltpu.SemaphoreType.DMA,
            pltpu.SemaphoreType.DMA,
        ],
        compiler_params=pltpu.CompilerParams(
            use_tc_tiling_on_sc=False, needs_layout_passes=False
        ),
    )(src4, table)

    mid5 = mid.reshape(NW, HB, 8, D_MODEL, 128)
    out5 = pl.pallas_call(
        _tc_transpose_body,
        grid=(BB, HB),
        in_specs=[
            pl.BlockSpec((1, 1, 8, D_MODEL, 128), lambda bb, rg: (bb, rg, 0, 0, 0))
        ],
        out_specs=pl.BlockSpec(
            (8, 8, 1, 8, 128), lambda bb, rg: (rg, 0, bb, 0, 0)
        ),
        out_shape=jax.ShapeDtypeStruct((200, 8, BB, 8, 128), jnp.float32),
        compiler_params=pltpu.CompilerParams(
            dimension_semantics=("arbitrary", "arbitrary")
        ),
    )(mid5)
    return out5.transpose(2, 4, 0, 1, 3).reshape(4096, 200, D_MODEL)


# speedup vs baseline: 10.1739x; 10.1739x over previous
import jax
import jax.numpy as jnp
from jax import lax
from jax.experimental import pallas as pl
from jax.experimental.pallas import tpu as pltpu
from jax.experimental.pallas import tpu_sc as plsc

D_MODEL = 64
SCALE = 8.0
NC, NS = 2, 16
NW = NC * NS
L = 16
HB = 200 // 8
BB = 4096 // 128
H_SB = 4
TPAD = 129


def _encoder_body(src4_hbm, table_hbm, out5_hbm, idx_all, rows2, trans_v, sg0, sg1, ss):
    wid = lax.axis_index("s") * NC + lax.axis_index("c")
    sg = (sg0, sg1)
    rowvecs = [
        [lax.iota(jnp.int32, L) + (j * D_MODEL + k * L) for k in range(D_MODEL // L)]
        for j in range(H_SB)
    ]

    pltpu.sync_copy(src4_hbm.at[pl.ds(0, HB), wid], idx_all)

    def fire_gather(k, half, b):
        for j in range(H_SB):
            pltpu.async_copy(
                table_hbm.at[idx_all.at[k, half * H_SB + j]],
                rows2.at[b, pl.ds(j * 128, 128)],
                sg[b],
            )

    def wait_gather(b):
        pltpu.make_async_copy(
            table_hbm.at[pl.ds(0, H_SB * 128)], rows2.at[b], sg[b]
        ).wait()

    def transpose_scale(b):
        for j in range(H_SB):
            @pl.loop(0, 128, init_carry=jnp.zeros((L,), jnp.int32), unroll=8)
            def _(bs, bvec):
                for k in range(D_MODEL // L):
                    val = rows2[b, j * 128 + bs, pl.ds(k * L, L)] * SCALE
                    plsc.store_scatter(trans_v, [rowvecs[j][k], bvec], val)
                return bvec + 1

    def fire_stores(k, half):
        h0 = k * 8 + half * H_SB
        for j in range(H_SB):
            for db in range(8):
                pltpu.async_copy(
                    trans_v.at[pl.ds(j * D_MODEL + db * 8, 8), pl.ds(0, 128)],
                    out5_hbm.at[h0 + j, db, wid],
                    ss,
                )

    def wait_stores():
        for j in range(H_SB):
            for db in range(8):
                pltpu.make_async_copy(
                    trans_v.at[pl.ds(j * D_MODEL + db * 8, 8), pl.ds(0, 128)],
                    out5_hbm.at[0, db, wid],
                    ss,
                ).wait()

    fire_gather(0, 0, 0)
    fire_gather(0, 1, 1)

    @pl.loop(0, HB)
    def _pair(k):
        wait_gather(0)

        @pl.when(k > 0)
        def _():
            wait_stores()

        transpose_scale(0)
        fire_stores(k, 0)

        @pl.when(k < HB - 1)
        def _():
            fire_gather(k + 1, 0, 0)

        wait_gather(1)
        wait_stores()
        transpose_scale(1)
        fire_stores(k, 1)

        @pl.when(k < HB - 1)
        def _():
            fire_gather(k + 1, 1, 1)

    wait_stores()


def kernel(src, table):
    src4 = (
        src.astype(jnp.int32)
        .swapaxes(0, 1)
        .reshape(HB, 8, BB, 128)
        .transpose(0, 2, 1, 3)
    )
    mesh = plsc.VectorSubcoreMesh(
        core_axis_name="c", subcore_axis_name="s", num_cores=NC, num_subcores=NS
    )
    out5 = pl.kernel(
        _encoder_body,
        out_type=jax.ShapeDtypeStruct((200, 8, BB, 8, 128), jnp.float32),
        mesh=mesh,
        scratch_types=[
            pltpu.VMEM((HB, 8, 128), jnp.int32),
            pltpu.VMEM((2, H_SB * 128, D_MODEL), jnp.float32),
            pltpu.VMEM((H_SB * D_MODEL, TPAD), jnp.float32),
            pltpu.SemaphoreType.DMA,
            pltpu.SemaphoreType.DMA,
            pltpu.SemaphoreType.DMA,
        ],
        compiler_params=pltpu.CompilerParams(
            use_tc_tiling_on_sc=False, needs_layout_passes=False
        ),
    )(src4, table)
    return out5.transpose(2, 4, 0, 1, 3).reshape(4096, 200, D_MODEL)
